# baseline (speedup 1.0000x reference)
import functools

import jax
import jax.numpy as jnp
from jax import lax
from jax.experimental import pallas as pl
from jax.experimental.pallas import tpu as pltpu
from jax.experimental.pallas import tpu_sc as plsc

C = 64
B, N, Q, D = 16, 2048, 2048, 512
QT = 2048
NQ = Q // QT

NC, NS = 2, 16
NW = NC * NS
CHUNK = 64
NCHUNK = N // CHUNK
DCOL = D // 2
EPC = NCHUNK

_sc_mesh = plsc.VectorSubcoreMesh(core_axis_name="c", subcore_axis_name="s")


@functools.partial(
    pl.kernel,
    out_type=jax.ShapeDtypeStruct((B * C, D), jnp.float32),
    mesh=_sc_mesh,
    compiler_params=pltpu.CompilerParams(needs_layout_passes=False),
    scratch_types=[
        pltpu.VMEM((NCHUNK, CHUNK), jnp.int32),
        pltpu.VMEM((CHUNK, DCOL), jnp.float32),
        pltpu.VMEM((CHUNK, DCOL), jnp.float32),
        pltpu.VMEM((CHUNK, DCOL), jnp.float32),
        pltpu.VMEM((CHUNK, DCOL), jnp.float32),
        pltpu.VMEM((C, DCOL), jnp.float32),
        pltpu.SemaphoreType.DMA,
        pltpu.SemaphoreType.DMA,
        pltpu.SemaphoreType.DMA,
        pltpu.SemaphoreType.DMA,
    ],
)
def _sc_proto_sums(sup_hbm, idx_hbm, out_hbm, idx_v, buf0, buf1, buf2, buf3,
                   acc, sem0, sem1, sem2, sem3):
    cid = lax.axis_index("c")
    sid = lax.axis_index("s")
    w = cid * NS + sid
    e = w // 2
    col0 = (w % 2) * DCOL
    row0 = e * N

    def _zero_row(i, _):
        for k in range(DCOL // 16):
            acc[i, pl.ds(k * 16, 16)] = jnp.zeros((16,), jnp.float32)
        return 0
    lax.fori_loop(0, C, _zero_row, 0)

    pltpu.sync_copy(idx_hbm.at[pl.ds(e * EPC, EPC)], idx_v)

    lanes = lax.iota(jnp.int32, 16)
    cols = [lanes + (k * 16) for k in range(DCOL // 16)]

    def _start(j, buf, sem):
        pltpu.async_copy(
            sup_hbm.at[pl.ds(row0 + j * CHUNK, CHUNK), pl.ds(col0, DCOL)],
            buf, sem)

    def _wait(buf, sem):
        pltpu.make_async_copy(
            sup_hbm.at[pl.ds(row0, CHUNK), pl.ds(col0, DCOL)], buf,
            sem).wait()

    def _scatter_chunk(j, buf):
        def _grp(g, _):
            tv = idx_v[j, pl.ds(g * 16, 16)]
            for i in range(16):
                trow = jnp.full((16,), tv[i], jnp.int32)
                r = g * 16 + i
                for k in range(DCOL // 16):
                    plsc.addupdate_scatter(
                        acc, [trow, cols[k]],
                        buf[r, pl.ds(k * 16, 16)])
            return 0
        lax.fori_loop(0, CHUNK // 16, _grp, 0)

    bufs = (buf0, buf1, buf2, buf3)
    sems = (sem0, sem1, sem2, sem3)
    for i in range(4):
        _start(i, bufs[i], sems[i])

    def _quad(g, _):
        for i in range(4):
            _wait(bufs[i], sems[i])
            _scatter_chunk(4 * g + i, bufs[i])
            _start(jnp.minimum(4 * g + 4 + i, NCHUNK - 4 + i),
                   bufs[i], sems[i])
        return 0
    lax.fori_loop(0, NCHUNK // 4, _quad, 0)

    for i in range(4):
        _wait(bufs[i], sems[i])

    pltpu.sync_copy(acc, out_hbm.at[pl.ds(e * C, C), pl.ds(col0, DCOL)])


def _tc_body(sup_t3_ref, q_ref, qt3_ref, sums_ref,
             dist_ref, pred_ref, loss_ref, acc_ref,
             protos_ref, p2_ref):
    b = pl.program_id(0)
    qi = pl.program_id(1)

    @pl.when(qi == 0)
    def _compute_protos():
        t = sup_t3_ref[0, 0, :]
        onehot = (lax.broadcasted_iota(jnp.int32, (C, N), 0)
                  == t[None, :]).astype(jnp.float32)
        counts = jnp.sum(onehot, axis=1, keepdims=True)
        protos = sums_ref[0] / jnp.maximum(counts, 1.0)
        protos_ref[...] = protos
        p2_ref[...] = jnp.sum(protos * protos, axis=1, keepdims=True)

    qblk = q_ref[0]
    q2 = jnp.sum(qblk * qblk, axis=1)
    protos = protos_ref[...]
    cross = lax.dot_general(protos, qblk, (((1,), (1,)), ((), ())),
                            preferred_element_type=jnp.float32)
    dist = p2_ref[...] + q2[None, :] - 2.0 * cross
    dist_ref[0] = dist

    logits = -dist
    mx = jnp.max(logits, axis=0, keepdims=True)
    se = jnp.sum(jnp.exp(logits - mx), axis=0, keepdims=True)
    lse = mx + jnp.log(se)
    tq = qt3_ref[0, 0, :]
    cls_iota = lax.broadcasted_iota(jnp.int32, (C, QT), 0)
    sel = jnp.sum(jnp.where(cls_iota == tq[None, :], logits, 0.0),
                  axis=0, keepdims=True)
    nll_sum = jnp.sum(lse - sel)

    mn = jnp.min(dist, axis=0, keepdims=True)
    pred = jnp.min(jnp.where(dist == mn, cls_iota, C), axis=0)
    pred_ref[0, 0, :] = pred
    acc_sum = jnp.sum((pred == tq).astype(jnp.float32))

    @pl.when((b == 0) & (qi == 0))
    def _init_stats():
        loss_ref[...] = jnp.zeros_like(loss_ref)
        acc_ref[...] = jnp.zeros_like(acc_ref)

    inv = 1.0 / (B * Q)
    loss_ref[...] += jnp.full((1, 128), nll_sum * inv, jnp.float32)
    acc_ref[...] += jnp.full((1, 128), acc_sum * inv, jnp.float32)


def kernel(support_embeddings, support_targets, query_embeddings,
           query_targets):
    sums = _sc_proto_sums(
        support_embeddings.reshape(B * N, D),
        support_targets.astype(jnp.int32).reshape(B * N // CHUNK, CHUNK))
    sums3 = sums.reshape(B, C, D)

    sup_t3 = support_targets.reshape(B, 1, N)
    qt3 = query_targets.reshape(B * NQ, 1, QT)

    grid = (B, NQ)
    dist, pred3, loss_v, acc_v = pl.pallas_call(
        _tc_body,
        grid=grid,
        in_specs=[
            pl.BlockSpec((1, 1, N), lambda b, q: (b, 0, 0)),
            pl.BlockSpec((1, QT, D), lambda b, q: (b, q, 0)),
            pl.BlockSpec((1, 1, QT), lambda b, q: (b * NQ + q, 0, 0)),
            pl.BlockSpec((1, C, D), lambda b, q: (b, 0, 0)),
        ],
        out_specs=[
            pl.BlockSpec((1, C, QT), lambda b, q: (b, 0, q)),
            pl.BlockSpec((1, 1, QT), lambda b, q: (b * NQ + q, 0, 0)),
            pl.BlockSpec((1, 128), lambda b, q: (0, 0)),
            pl.BlockSpec((1, 128), lambda b, q: (0, 0)),
        ],
        out_shape=[
            jax.ShapeDtypeStruct((B, C, Q), jnp.float32),
            jax.ShapeDtypeStruct((B * NQ, 1, QT), jnp.int32),
            jax.ShapeDtypeStruct((1, 128), jnp.float32),
            jax.ShapeDtypeStruct((1, 128), jnp.float32),
        ],
        scratch_shapes=[
            pltpu.VMEM((C, D), jnp.float32),
            pltpu.VMEM((C, 1), jnp.float32),
        ],
    )(sup_t3, query_embeddings, qt3, sums3)

    predictions = pred3.reshape(B, Q)
    loss = loss_v[0, 0]
    accuracy = acc_v[0, 0]
    return (loss, predictions, accuracy, dist)


# speedup vs baseline: 1.0350x; 1.0350x over previous
import functools

import jax
import jax.numpy as jnp
from jax import lax
from jax.experimental import pallas as pl
from jax.experimental.pallas import tpu as pltpu
from jax.experimental.pallas import tpu_sc as plsc

C = 64
B, N, Q, D = 16, 2048, 2048, 512
QT = 2048
NQ = Q // QT

NC, NS = 2, 16
NW = NC * NS
CHUNK = 64
NCHUNK = N // CHUNK
DCOL = D // 2
EPC = NCHUNK

_sc_mesh = plsc.VectorSubcoreMesh(core_axis_name="c", subcore_axis_name="s")


@functools.partial(
    pl.kernel,
    out_type=jax.ShapeDtypeStruct((B * C, D), jnp.float32),
    mesh=_sc_mesh,
    compiler_params=pltpu.CompilerParams(needs_layout_passes=False),
    scratch_types=[
        pltpu.VMEM((NCHUNK, CHUNK), jnp.int32),
        pltpu.VMEM((CHUNK, DCOL), jnp.float32),
        pltpu.VMEM((CHUNK, DCOL), jnp.float32),
        pltpu.VMEM((C, DCOL), jnp.float32),
        pltpu.SemaphoreType.DMA,
        pltpu.SemaphoreType.DMA,
    ],
)
def _sc_proto_sums(sup_hbm, idx_hbm, out_hbm, idx_v, buf_a, buf_b, acc,
                   sem_a, sem_b):
    cid = lax.axis_index("c")
    sid = lax.axis_index("s")
    w = cid * NS + sid
    e = w // 2
    col0 = (w % 2) * DCOL
    row0 = e * N

    def _zero_row(i, _):
        for k in range(DCOL // 16):
            acc[i, pl.ds(k * 16, 16)] = jnp.zeros((16,), jnp.float32)
        return 0
    lax.fori_loop(0, C, _zero_row, 0)

    pltpu.sync_copy(idx_hbm.at[pl.ds(e * EPC, EPC)], idx_v)

    lanes = lax.iota(jnp.int32, 16)
    cols = [lanes + (k * 16) for k in range(DCOL // 16)]

    def _start(j, buf, sem):
        pltpu.async_copy(
            sup_hbm.at[pl.ds(row0 + j * CHUNK, CHUNK), pl.ds(col0, DCOL)],
            buf, sem)

    def _wait(buf, sem):
        pltpu.make_async_copy(
            sup_hbm.at[pl.ds(row0, CHUNK), pl.ds(col0, DCOL)], buf,
            sem).wait()

    def _scatter_chunk(j, buf):
        def _grp(g, _):
            tv = idx_v[j, pl.ds(g * 16, 16)]
            for i in range(16):
                trow = jnp.full((16,), tv[i], jnp.int32)
                r = g * 16 + i
                for k in range(DCOL // 16):
                    plsc.addupdate_scatter(
                        acc, [trow, cols[k]],
                        buf[r, pl.ds(k * 16, 16)])
            return 0
        lax.fori_loop(0, CHUNK // 16, _grp, 0)

    _start(0, buf_a, sem_a)
    _start(1, buf_b, sem_b)

    def _pair(g, _):
        _wait(buf_a, sem_a)
        _scatter_chunk(2 * g, buf_a)
        _start(jnp.minimum(2 * g + 2, NCHUNK - 2), buf_a, sem_a)
        _wait(buf_b, sem_b)
        _scatter_chunk(2 * g + 1, buf_b)
        _start(jnp.minimum(2 * g + 3, NCHUNK - 1), buf_b, sem_b)
        return 0
    lax.fori_loop(0, NCHUNK // 2, _pair, 0)

    _wait(buf_a, sem_a)
    _wait(buf_b, sem_b)

    pltpu.sync_copy(acc, out_hbm.at[pl.ds(e * C, C), pl.ds(col0, DCOL)])


def _tc_body(sup_t3_ref, q_ref, qt3_ref, sums_ref,
             dist_ref, pred_ref, loss_ref, acc_ref,
             protos_ref, p2_ref):
    b = pl.program_id(0)
    qi = pl.program_id(1)

    @pl.when(qi == 0)
    def _compute_protos():
        t = sup_t3_ref[0, 0, :]
        onehot = (lax.broadcasted_iota(jnp.int32, (C, N), 0)
                  == t[None, :]).astype(jnp.float32)
        counts = jnp.sum(onehot, axis=1, keepdims=True)
        protos = sums_ref[0] / jnp.maximum(counts, 1.0)
        protos_ref[...] = protos
        p2_ref[...] = jnp.sum(protos * protos, axis=1, keepdims=True)

    qblk = q_ref[0]
    q2 = jnp.sum(qblk * qblk, axis=1)
    protos = protos_ref[...]
    cross = lax.dot_general(protos, qblk, (((1,), (1,)), ((), ())),
                            preferred_element_type=jnp.float32)
    dist = p2_ref[...] + q2[None, :] - 2.0 * cross
    dist_ref[0] = dist

    logits = -dist
    mx = jnp.max(logits, axis=0, keepdims=True)
    se = jnp.sum(jnp.exp(logits - mx), axis=0, keepdims=True)
    lse = mx + jnp.log(se)
    tq = qt3_ref[0, 0, :]
    cls_iota = lax.broadcasted_iota(jnp.int32, (C, QT), 0)
    sel = jnp.sum(jnp.where(cls_iota == tq[None, :], logits, 0.0),
                  axis=0, keepdims=True)
    nll_sum = jnp.sum(lse - sel)

    mn = jnp.min(dist, axis=0, keepdims=True)
    pred = jnp.min(jnp.where(dist == mn, cls_iota, C), axis=0)
    pred_ref[0, 0, :] = pred
    acc_sum = jnp.sum((pred == tq).astype(jnp.float32))

    @pl.when((b == 0) & (qi == 0))
    def _init_stats():
        loss_ref[...] = jnp.zeros_like(loss_ref)
        acc_ref[...] = jnp.zeros_like(acc_ref)

    inv = 1.0 / (B * Q)
    loss_ref[...] += jnp.full((1, 128), nll_sum * inv, jnp.float32)
    acc_ref[...] += jnp.full((1, 128), acc_sum * inv, jnp.float32)


def kernel(support_embeddings, support_targets, query_embeddings,
           query_targets):
    sums = _sc_proto_sums(
        support_embeddings.reshape(B * N, D),
        support_targets.astype(jnp.int32).reshape(B * N // CHUNK, CHUNK))
    sums3 = sums.reshape(B, C, D)

    sup_t3 = support_targets.reshape(B, 1, N)
    qt3 = query_targets.reshape(B * NQ, 1, QT)

    grid = (B, NQ)
    dist, pred3, loss_v, acc_v = pl.pallas_call(
        _tc_body,
        grid=grid,
        in_specs=[
            pl.BlockSpec((1, 1, N), lambda b, q: (b, 0, 0)),
            pl.BlockSpec((1, QT, D), lambda b, q: (b, q, 0)),
            pl.BlockSpec((1, 1, QT), lambda b, q: (b * NQ + q, 0, 0)),
            pl.BlockSpec((1, C, D), lambda b, q: (b, 0, 0)),
        ],
        out_specs=[
            pl.BlockSpec((1, C, QT), lambda b, q: (b, 0, q)),
            pl.BlockSpec((1, 1, QT), lambda b, q: (b * NQ + q, 0, 0)),
            pl.BlockSpec((1, 128), lambda b, q: (0, 0)),
            pl.BlockSpec((1, 128), lambda b, q: (0, 0)),
        ],
        out_shape=[
            jax.ShapeDtypeStruct((B, C, Q), jnp.float32),
            jax.ShapeDtypeStruct((B * NQ, 1, QT), jnp.int32),
            jax.ShapeDtypeStruct((1, 128), jnp.float32),
            jax.ShapeDtypeStruct((1, 128), jnp.float32),
        ],
        scratch_shapes=[
            pltpu.VMEM((C, D), jnp.float32),
            pltpu.VMEM((C, 1), jnp.float32),
        ],
    )(sup_t3, query_embeddings, qt3, sums3)

    predictions = pred3.reshape(B, Q)
    loss = loss_v[0, 0]
    accuracy = acc_v[0, 0]
    return (loss, predictions, accuracy, dist)
